# baseline (speedup 1.0000x reference)
import functools

import jax
import jax.numpy as jnp
from jax import lax
from jax.experimental import pallas as pl
from jax.experimental.pallas import tpu as pltpu
from jax.experimental.pallas import tpu_sc as plsc

TOK = 32768
DIM = 256
NSEG = 16
EPS = 1e-5

NW = 32
SC_TOK = 8192
CHUNK = SC_TOK // NW
BLK = 64
BLKP = BLK + 8
LANES = 16
NSTRIP = DIM // LANES
CU_PAD = 32


def _sc_body(x_hbm, cu_hbm, psum_hbm, psq_hbm, xbuf0, xbuf1, asum, asq,
             cu_v, sem0, sem1):
    wid = lax.axis_index("s") * 2 + lax.axis_index("c")
    base = wid * CHUNK
    pltpu.sync_copy(cu_hbm, cu_v)
    zeros = jnp.zeros((LANES,), jnp.float32)

    def cu_at(i):
        return cu_v[pl.ds(i, LANES)][0]

    def zero_body(s, carry):
        for k in range(NSTRIP):
            sl = pl.ds(k * LANES, LANES)
            asum[s, sl] = zeros
            asq[s, sl] = zeros
        return carry

    lax.fori_loop(0, NSEG, zero_body, jnp.int32(0))

    limit = base + CHUNK

    def seg_body(s, carry0):
        lo = jnp.maximum(cu_at(s), base)
        hi = jnp.minimum(cu_at(s + 1), limit)
        seg_len = jnp.maximum(hi - lo, 0)
        nblk = (seg_len + BLK - 1) >> 6

        def window(j):
            blk_lo = lo + j * BLK
            start = jnp.minimum(jnp.bitwise_and(blk_lo, -8), TOK - BLKP)
            return pl.multiple_of(start, 8)

        def accumulate(buf, j):
            blk_lo = lo + j * BLK
            start = window(j)
            loc_lo = blk_lo - start
            loc_hi = jnp.minimum(hi, blk_lo + BLK) - start

            def tok_body(t, c):
                out = []
                for k in range(NSTRIP):
                    row = buf[t, pl.ds(k * LANES, LANES)]
                    out.append(c[2 * k] + row)
                    out.append(c[2 * k + 1] + row * row)
                return tuple(out)

            acc = lax.fori_loop(loc_lo, jnp.maximum(loc_hi, loc_lo),
                                tok_body, (zeros,) * (2 * NSTRIP))
            for k in range(NSTRIP):
                sl = pl.ds(k * LANES, LANES)
                asum[s, sl] = asum[s, sl] + acc[2 * k]
                asq[s, sl] = asq[s, sl] + acc[2 * k + 1]

        def fetch(j, buf, sem):
            pltpu.async_copy(x_hbm.at[pl.ds(window(j), BLKP)], buf, sem)

        def wait(buf, sem):
            pltpu.make_async_copy(
                x_hbm.at[pl.ds(window(0), BLKP)], buf, sem).wait()

        @pl.when(nblk > 0)
        def _():
            fetch(0, xbuf0, sem0)
            npair = (nblk + 1) >> 1

            def pair_body(m, carry):
                j0 = 2 * m
                wait(xbuf0, sem0)
                fetch(jnp.minimum(j0 + 1, nblk - 1), xbuf1, sem1)
                accumulate(xbuf0, j0)
                wait(xbuf1, sem1)
                fetch(jnp.minimum(j0 + 2, nblk - 1), xbuf0, sem0)
                accumulate(xbuf1, j0 + 1)
                return carry

            lax.fori_loop(0, npair, pair_body, jnp.int32(0))
            wait(xbuf0, sem0)

        return carry0

    lax.fori_loop(0, NSEG, seg_body, jnp.int32(0))
    pltpu.sync_copy(asum, psum_hbm.at[wid])
    pltpu.sync_copy(asq, psq_hbm.at[wid])


_sc_partials = functools.partial(
    pl.kernel,
    out_type=(
        jax.ShapeDtypeStruct((NW, NSEG, DIM), jnp.float32),
        jax.ShapeDtypeStruct((NW, NSEG, DIM), jnp.float32),
    ),
    mesh=plsc.VectorSubcoreMesh(
        core_axis_name="c", subcore_axis_name="s",
        num_cores=2, num_subcores=16),
    scratch_types=[
        pltpu.VMEM((BLKP, DIM), jnp.float32),
        pltpu.VMEM((BLKP, DIM), jnp.float32),
        pltpu.VMEM((NSEG, DIM), jnp.float32),
        pltpu.VMEM((NSEG, DIM), jnp.float32),
        pltpu.VMEM((CU_PAD,), jnp.int32),
        pltpu.SemaphoreType.DMA,
        pltpu.SemaphoreType.DMA,
    ],
)(_sc_body)


PBLK = 4096
SC_BLKS = SC_TOK // PBLK
P1 = (TOK - SC_TOK) // PBLK
G2 = TOK // PBLK


def _seg_onehot(cu_smem, tok0, n):
    tok = tok0 + lax.broadcasted_iota(jnp.int32, (n, 1), 0)
    bid = jnp.zeros((n, 1), jnp.int32)
    for j in range(1, NSEG):
        bid += (tok >= cu_smem[j]).astype(jnp.int32)
    seg = lax.broadcasted_iota(jnp.int32, (1, NSEG), 1)
    return (bid == seg).astype(jnp.float32)


def _tc_fused_body(cu_smem, counts_ref, psum_ref, psq_ref, x_ref, o_ref,
                   stash, acc_s, acc_q, mean_sc, rstd_sc):
    pid = pl.program_id(0)

    @pl.when(pid < P1)
    def _():
        xs = x_ref[...]
        oh = _seg_onehot(cu_smem, SC_TOK + pid * PBLK, PBLK)
        dn = (((0,), (0,)), ((), ()))
        ps = lax.dot_general(oh, xs, dn,
                             preferred_element_type=jnp.float32)
        pq = lax.dot_general(oh, xs * xs, dn,
                             preferred_element_type=jnp.float32)
        stash[pl.ds(pid * PBLK, PBLK), :] = xs

        @pl.when(pid == 0)
        def _():
            acc_s[...] = ps
            acc_q[...] = pq

        @pl.when(pid > 0)
        def _():
            acc_s[...] = acc_s[...] + ps
            acc_q[...] = acc_q[...] + pq

    @pl.when(pid == P1)
    def _():
        s = jnp.sum(psum_ref[...], axis=0) + acc_s[...]
        q = jnp.sum(psq_ref[...], axis=0) + acc_q[...]
        cnt = counts_ref[...]
        mean = s / cnt
        var = jnp.maximum(q / cnt - mean * mean, 0.0)
        mean_sc[...] = mean
        rstd_sc[...] = lax.rsqrt(var + EPS)

    @pl.when(pid >= P1)
    def _():
        j = pid - P1
        oh = _seg_onehot(cu_smem, j * PBLK, PBLK)
        mu = jnp.dot(oh, mean_sc[...], preferred_element_type=jnp.float32)
        rs = jnp.dot(oh, rstd_sc[...], preferred_element_type=jnp.float32)

        @pl.when(j < SC_BLKS)
        def _():
            o_ref[...] = (x_ref[...] - mu) * rs

        @pl.when(j >= SC_BLKS)
        def _():
            xv = stash[pl.ds((j - SC_BLKS) * PBLK, PBLK), :]
            o_ref[...] = (xv - mu) * rs


def _x_index(i):
    return (jnp.where(i < P1, SC_BLKS + i,
                      jnp.minimum(i - P1, SC_BLKS - 1)), 0)


_tc_fused = pl.pallas_call(
    _tc_fused_body,
    grid=(P1 + G2,),
    in_specs=[
        pl.BlockSpec(memory_space=pltpu.SMEM),
        pl.BlockSpec((NSEG, 1), lambda i: (0, 0)),
        pl.BlockSpec((NW, NSEG, DIM), lambda i: (0, 0, 0)),
        pl.BlockSpec((NW, NSEG, DIM), lambda i: (0, 0, 0)),
        pl.BlockSpec((PBLK, DIM), _x_index),
    ],
    out_specs=pl.BlockSpec((PBLK, DIM),
                           lambda i: (jnp.maximum(i - P1, 0), 0)),
    out_shape=jax.ShapeDtypeStruct((TOK, DIM), jnp.float32),
    scratch_shapes=[
        pltpu.VMEM((TOK - SC_TOK, DIM), jnp.float32),
        pltpu.VMEM((NSEG, DIM), jnp.float32),
        pltpu.VMEM((NSEG, DIM), jnp.float32),
        pltpu.VMEM((NSEG, DIM), jnp.float32),
        pltpu.VMEM((NSEG, DIM), jnp.float32),
    ],
)


def kernel(x, cu_seqlens):
    cu = cu_seqlens.astype(jnp.int32)
    cu32 = jnp.concatenate(
        [cu, jnp.full((CU_PAD - NSEG - 1,), TOK, jnp.int32)])
    psum, psq = _sc_partials(x, cu32)
    counts = jnp.maximum(
        (cu[1:] - cu[:-1]).astype(jnp.float32), 1.0).reshape(NSEG, 1)
    return _tc_fused(cu32, counts, psum, psq, x)


# speedup vs baseline: 1.2212x; 1.2212x over previous
import functools

import jax
import jax.numpy as jnp
from jax import lax
from jax.experimental import pallas as pl
from jax.experimental.pallas import tpu as pltpu
from jax.experimental.pallas import tpu_sc as plsc

TOK = 32768
DIM = 256
NSEG = 16
EPS = 1e-5

NW = 32
SC_TOK = 6144
CHUNK = SC_TOK // NW
BLK = 64
BLKP = BLK + 8
LANES = 16
NSTRIP = DIM // LANES
CU_PAD = 32


def _sc_body(x_hbm, cu_hbm, psum_hbm, psq_hbm, xbuf0, xbuf1, asum, asq,
             cu_v, sem0, sem1):
    wid = lax.axis_index("s") * 2 + lax.axis_index("c")
    base = wid * CHUNK
    pltpu.sync_copy(cu_hbm, cu_v)
    zeros = jnp.zeros((LANES,), jnp.float32)

    def cu_at(i):
        return cu_v[pl.ds(i, LANES)][0]

    def zero_body(s, carry):
        for k in range(NSTRIP):
            sl = pl.ds(k * LANES, LANES)
            asum[s, sl] = zeros
            asq[s, sl] = zeros
        return carry

    lax.fori_loop(0, NSEG, zero_body, jnp.int32(0))

    limit = base + CHUNK

    def seg_body(s, carry0):
        lo = jnp.maximum(cu_at(s), base)
        hi = jnp.minimum(cu_at(s + 1), limit)
        seg_len = jnp.maximum(hi - lo, 0)
        nblk = (seg_len + BLK - 1) >> 6

        def window(j):
            blk_lo = lo + j * BLK
            start = jnp.minimum(jnp.bitwise_and(blk_lo, -8), TOK - BLKP)
            return pl.multiple_of(start, 8)

        def accumulate(buf, j):
            blk_lo = lo + j * BLK
            start = window(j)
            loc_lo = blk_lo - start
            loc_hi = jnp.minimum(hi, blk_lo + BLK) - start

            def tok_body(t, c):
                out = []
                for k in range(NSTRIP):
                    row = buf[t, pl.ds(k * LANES, LANES)]
                    out.append(c[2 * k] + row)
                    out.append(c[2 * k + 1] + row * row)
                return tuple(out)

            acc = lax.fori_loop(loc_lo, jnp.maximum(loc_hi, loc_lo),
                                tok_body, (zeros,) * (2 * NSTRIP))
            for k in range(NSTRIP):
                sl = pl.ds(k * LANES, LANES)
                asum[s, sl] = asum[s, sl] + acc[2 * k]
                asq[s, sl] = asq[s, sl] + acc[2 * k + 1]

        def fetch(j, buf, sem):
            pltpu.async_copy(x_hbm.at[pl.ds(window(j), BLKP)], buf, sem)

        def wait(buf, sem):
            pltpu.make_async_copy(
                x_hbm.at[pl.ds(window(0), BLKP)], buf, sem).wait()

        @pl.when(nblk > 0)
        def _():
            fetch(0, xbuf0, sem0)
            npair = (nblk + 1) >> 1

            def pair_body(m, carry):
                j0 = 2 * m
                wait(xbuf0, sem0)
                fetch(jnp.minimum(j0 + 1, nblk - 1), xbuf1, sem1)
                accumulate(xbuf0, j0)
                wait(xbuf1, sem1)
                fetch(jnp.minimum(j0 + 2, nblk - 1), xbuf0, sem0)
                accumulate(xbuf1, j0 + 1)
                return carry

            lax.fori_loop(0, npair, pair_body, jnp.int32(0))
            wait(xbuf0, sem0)

        return carry0

    lax.fori_loop(0, NSEG, seg_body, jnp.int32(0))
    pltpu.sync_copy(asum, psum_hbm.at[wid])
    pltpu.sync_copy(asq, psq_hbm.at[wid])


_sc_partials = functools.partial(
    pl.kernel,
    out_type=(
        jax.ShapeDtypeStruct((NW, NSEG, DIM), jnp.float32),
        jax.ShapeDtypeStruct((NW, NSEG, DIM), jnp.float32),
    ),
    mesh=plsc.VectorSubcoreMesh(
        core_axis_name="c", subcore_axis_name="s",
        num_cores=2, num_subcores=16),
    scratch_types=[
        pltpu.VMEM((BLKP, DIM), jnp.float32),
        pltpu.VMEM((BLKP, DIM), jnp.float32),
        pltpu.VMEM((NSEG, DIM), jnp.float32),
        pltpu.VMEM((NSEG, DIM), jnp.float32),
        pltpu.VMEM((CU_PAD,), jnp.int32),
        pltpu.SemaphoreType.DMA,
        pltpu.SemaphoreType.DMA,
    ],
)(_sc_body)


PBLK = 3328
PGRID = (TOK - SC_TOK) // PBLK


def _seg_onehot(cu_smem, tok0, n):
    tok = tok0 + lax.broadcasted_iota(jnp.int32, (n, 1), 0)
    bid = jnp.zeros((n, 1), jnp.int32)
    for j in range(1, NSEG):
        bid += (tok >= cu_smem[j]).astype(jnp.int32)
    seg = lax.broadcasted_iota(jnp.int32, (1, NSEG), 1)
    return (bid == seg).astype(jnp.float32)


def _tc_part_body(cu_smem, x_ref, ps_ref, pq_ref, acc_s, acc_q):
    pid = pl.program_id(0)
    oh = _seg_onehot(cu_smem, SC_TOK + pid * PBLK, PBLK)
    xs = x_ref[...]
    dn = (((0,), (0,)), ((), ()))
    ps = lax.dot_general(oh, xs, dn, preferred_element_type=jnp.float32)
    pq = lax.dot_general(oh, xs * xs, dn,
                         preferred_element_type=jnp.float32)

    @pl.when(pid == 0)
    def _():
        acc_s[...] = ps
        acc_q[...] = pq

    @pl.when(pid > 0)
    def _():
        acc_s[...] = acc_s[...] + ps
        acc_q[...] = acc_q[...] + pq

    @pl.when(pid == PGRID - 1)
    def _():
        ps_ref[...] = acc_s[...]
        pq_ref[...] = acc_q[...]


_tc_partials = pl.pallas_call(
    _tc_part_body,
    grid=(PGRID,),
    in_specs=[
        pl.BlockSpec(memory_space=pltpu.SMEM),
        pl.BlockSpec((PBLK, DIM), lambda i: (i + SC_TOK // PBLK, 0)),
    ],
    out_specs=(
        pl.BlockSpec((NSEG, DIM), lambda i: (0, 0)),
        pl.BlockSpec((NSEG, DIM), lambda i: (0, 0)),
    ),
    out_shape=(
        jax.ShapeDtypeStruct((NSEG, DIM), jnp.float32),
        jax.ShapeDtypeStruct((NSEG, DIM), jnp.float32),
    ),
    scratch_shapes=[
        pltpu.VMEM((NSEG, DIM), jnp.float32),
        pltpu.VMEM((NSEG, DIM), jnp.float32),
    ],
)


TBLK = 4096
GRID = TOK // TBLK


def _tc_body(cu_smem, counts_ref, psum_ref, psq_ref, tps_ref, tpq_ref,
             x_ref, o_ref, mean_sc, rstd_sc):
    pid = pl.program_id(0)

    @pl.when(pid == 0)
    def _():
        s = jnp.sum(psum_ref[...], axis=0) + tps_ref[...]
        q = jnp.sum(psq_ref[...], axis=0) + tpq_ref[...]
        cnt = counts_ref[...]
        mean = s / cnt
        var = jnp.maximum(q / cnt - mean * mean, 0.0)
        mean_sc[...] = mean
        rstd_sc[...] = lax.rsqrt(var + EPS)

    oh = _seg_onehot(cu_smem, pid * TBLK, TBLK)
    mu = jnp.dot(oh, mean_sc[...], preferred_element_type=jnp.float32)
    rs = jnp.dot(oh, rstd_sc[...], preferred_element_type=jnp.float32)
    o_ref[...] = (x_ref[...] - mu) * rs


_tc_norm = pl.pallas_call(
    _tc_body,
    grid=(GRID,),
    in_specs=[
        pl.BlockSpec(memory_space=pltpu.SMEM),
        pl.BlockSpec((NSEG, 1), lambda i: (0, 0)),
        pl.BlockSpec((NW, NSEG, DIM), lambda i: (0, 0, 0)),
        pl.BlockSpec((NW, NSEG, DIM), lambda i: (0, 0, 0)),
        pl.BlockSpec((NSEG, DIM), lambda i: (0, 0)),
        pl.BlockSpec((NSEG, DIM), lambda i: (0, 0)),
        pl.BlockSpec((TBLK, DIM), lambda i: (i, 0)),
    ],
    out_specs=pl.BlockSpec((TBLK, DIM), lambda i: (i, 0)),
    out_shape=jax.ShapeDtypeStruct((TOK, DIM), jnp.float32),
    scratch_shapes=[
        pltpu.VMEM((NSEG, DIM), jnp.float32),
        pltpu.VMEM((NSEG, DIM), jnp.float32),
    ],
)


def kernel(x, cu_seqlens):
    cu = cu_seqlens.astype(jnp.int32)
    cu32 = jnp.concatenate(
        [cu, jnp.full((CU_PAD - NSEG - 1,), TOK, jnp.int32)])
    psum, psq = _sc_partials(x, cu32)
    tps, tpq = _tc_partials(cu32, x)
    counts = jnp.maximum(
        (cu[1:] - cu[:-1]).astype(jnp.float32), 1.0).reshape(NSEG, 1)
    return _tc_norm(cu32, counts, psum, psq, tps, tpq, x)


# speedup vs baseline: 1.2504x; 1.0239x over previous
import functools

import jax
import jax.numpy as jnp
from jax import lax
from jax.experimental import pallas as pl
from jax.experimental.pallas import tpu as pltpu
from jax.experimental.pallas import tpu_sc as plsc

TOK = 32768
DIM = 256
NSEG = 16
EPS = 1e-5

NW = 32
SC_TOK = 8192
CHUNK = SC_TOK // NW
BLK = 64
BLKP = BLK + 8
LANES = 16
NSTRIP = DIM // LANES
CU_PAD = 32


def _sc_body(x_hbm, cu_hbm, psum_hbm, psq_hbm, xbuf0, xbuf1, asum, asq,
             cu_v, sem0, sem1):
    wid = lax.axis_index("s") * 2 + lax.axis_index("c")
    base = wid * CHUNK
    pltpu.sync_copy(cu_hbm, cu_v)
    zeros = jnp.zeros((LANES,), jnp.float32)

    def cu_at(i):
        return cu_v[pl.ds(i, LANES)][0]

    def zero_body(s, carry):
        for k in range(NSTRIP):
            sl = pl.ds(k * LANES, LANES)
            asum[s, sl] = zeros
            asq[s, sl] = zeros
        return carry

    lax.fori_loop(0, NSEG, zero_body, jnp.int32(0))

    limit = base + CHUNK

    def seg_body(s, carry0):
        lo = jnp.maximum(cu_at(s), base)
        hi = jnp.minimum(cu_at(s + 1), limit)
        seg_len = jnp.maximum(hi - lo, 0)
        nblk = (seg_len + BLK - 1) >> 6

        def window(j):
            blk_lo = lo + j * BLK
            start = jnp.minimum(jnp.bitwise_and(blk_lo, -8), TOK - BLKP)
            return pl.multiple_of(start, 8)

        def accumulate(buf, j):
            blk_lo = lo + j * BLK
            start = window(j)
            loc_lo = blk_lo - start
            loc_hi = jnp.minimum(hi, blk_lo + BLK) - start

            def tok_body(t, c):
                out = []
                for k in range(NSTRIP):
                    row = buf[t, pl.ds(k * LANES, LANES)]
                    out.append(c[2 * k] + row)
                    out.append(c[2 * k + 1] + row * row)
                return tuple(out)

            acc = lax.fori_loop(loc_lo, jnp.maximum(loc_hi, loc_lo),
                                tok_body, (zeros,) * (2 * NSTRIP))
            for k in range(NSTRIP):
                sl = pl.ds(k * LANES, LANES)
                asum[s, sl] = asum[s, sl] + acc[2 * k]
                asq[s, sl] = asq[s, sl] + acc[2 * k + 1]

        def fetch(j, buf, sem):
            pltpu.async_copy(x_hbm.at[pl.ds(window(j), BLKP)], buf, sem)

        def wait(buf, sem):
            pltpu.make_async_copy(
                x_hbm.at[pl.ds(window(0), BLKP)], buf, sem).wait()

        @pl.when(nblk > 0)
        def _():
            fetch(0, xbuf0, sem0)
            npair = (nblk + 1) >> 1

            def pair_body(m, carry):
                j0 = 2 * m
                wait(xbuf0, sem0)
                fetch(jnp.minimum(j0 + 1, nblk - 1), xbuf1, sem1)
                accumulate(xbuf0, j0)
                wait(xbuf1, sem1)
                fetch(jnp.minimum(j0 + 2, nblk - 1), xbuf0, sem0)
                accumulate(xbuf1, j0 + 1)
                return carry

            lax.fori_loop(0, npair, pair_body, jnp.int32(0))
            wait(xbuf0, sem0)

        return carry0

    lax.fori_loop(0, NSEG, seg_body, jnp.int32(0))
    pltpu.sync_copy(asum, psum_hbm.at[wid])
    pltpu.sync_copy(asq, psq_hbm.at[wid])


_sc_partials = functools.partial(
    pl.kernel,
    out_type=(
        jax.ShapeDtypeStruct((NW, NSEG, DIM), jnp.float32),
        jax.ShapeDtypeStruct((NW, NSEG, DIM), jnp.float32),
    ),
    mesh=plsc.VectorSubcoreMesh(
        core_axis_name="c", subcore_axis_name="s",
        num_cores=2, num_subcores=16),
    scratch_types=[
        pltpu.VMEM((BLKP, DIM), jnp.float32),
        pltpu.VMEM((BLKP, DIM), jnp.float32),
        pltpu.VMEM((NSEG, DIM), jnp.float32),
        pltpu.VMEM((NSEG, DIM), jnp.float32),
        pltpu.VMEM((CU_PAD,), jnp.int32),
        pltpu.SemaphoreType.DMA,
        pltpu.SemaphoreType.DMA,
    ],
)(_sc_body)


PBLK = 4096
PGRID = (TOK - SC_TOK) // PBLK


def _seg_onehot(cu_smem, tok0, n):
    tok = tok0 + lax.broadcasted_iota(jnp.int32, (n, 1), 0)
    bid = jnp.zeros((n, 1), jnp.int32)
    for j in range(1, NSEG):
        bid += (tok >= cu_smem[j]).astype(jnp.int32)
    seg = lax.broadcasted_iota(jnp.int32, (1, NSEG), 1)
    return (bid == seg).astype(jnp.float32)


def _tc_part_body(cu_smem, x_ref, ps_ref, pq_ref, acc_s, acc_q):
    pid = pl.program_id(0)
    oh = _seg_onehot(cu_smem, SC_TOK + pid * PBLK, PBLK)
    xs = x_ref[...]
    dn = (((0,), (0,)), ((), ()))
    ps = lax.dot_general(oh, xs, dn, preferred_element_type=jnp.float32)
    pq = lax.dot_general(oh, xs * xs, dn,
                         preferred_element_type=jnp.float32)

    @pl.when(pid == 0)
    def _():
        acc_s[...] = ps
        acc_q[...] = pq

    @pl.when(pid > 0)
    def _():
        acc_s[...] = acc_s[...] + ps
        acc_q[...] = acc_q[...] + pq

    @pl.when(pid == PGRID - 1)
    def _():
        ps_ref[...] = acc_s[...]
        pq_ref[...] = acc_q[...]


_tc_partials = pl.pallas_call(
    _tc_part_body,
    grid=(PGRID,),
    in_specs=[
        pl.BlockSpec(memory_space=pltpu.SMEM),
        pl.BlockSpec((PBLK, DIM), lambda i: (i + SC_TOK // PBLK, 0)),
    ],
    out_specs=(
        pl.BlockSpec((NSEG, DIM), lambda i: (0, 0)),
        pl.BlockSpec((NSEG, DIM), lambda i: (0, 0)),
    ),
    out_shape=(
        jax.ShapeDtypeStruct((NSEG, DIM), jnp.float32),
        jax.ShapeDtypeStruct((NSEG, DIM), jnp.float32),
    ),
    scratch_shapes=[
        pltpu.VMEM((NSEG, DIM), jnp.float32),
        pltpu.VMEM((NSEG, DIM), jnp.float32),
    ],
)


TBLK = 4096
GRID = TOK // TBLK


def _tc_body(cu_smem, counts_ref, psum_ref, psq_ref, tps_ref, tpq_ref,
             x_ref, o_ref, mean_sc, rstd_sc):
    pid = pl.program_id(0)

    @pl.when(pid == 0)
    def _():
        s = jnp.sum(psum_ref[...], axis=0) + tps_ref[...]
        q = jnp.sum(psq_ref[...], axis=0) + tpq_ref[...]
        cnt = counts_ref[...]
        mean = s / cnt
        var = jnp.maximum(q / cnt - mean * mean, 0.0)
        mean_sc[...] = mean
        rstd_sc[...] = lax.rsqrt(var + EPS)

    oh = _seg_onehot(cu_smem, pid * TBLK, TBLK)
    mu = jnp.dot(oh, mean_sc[...], preferred_element_type=jnp.float32)
    rs = jnp.dot(oh, rstd_sc[...], preferred_element_type=jnp.float32)
    o_ref[...] = (x_ref[...] - mu) * rs


_tc_norm = pl.pallas_call(
    _tc_body,
    grid=(GRID,),
    in_specs=[
        pl.BlockSpec(memory_space=pltpu.SMEM),
        pl.BlockSpec((NSEG, 1), lambda i: (0, 0)),
        pl.BlockSpec((NW, NSEG, DIM), lambda i: (0, 0, 0)),
        pl.BlockSpec((NW, NSEG, DIM), lambda i: (0, 0, 0)),
        pl.BlockSpec((NSEG, DIM), lambda i: (0, 0)),
        pl.BlockSpec((NSEG, DIM), lambda i: (0, 0)),
        pl.BlockSpec((TBLK, DIM), lambda i: (i, 0)),
    ],
    out_specs=pl.BlockSpec((TBLK, DIM), lambda i: (i, 0)),
    out_shape=jax.ShapeDtypeStruct((TOK, DIM), jnp.float32),
    scratch_shapes=[
        pltpu.VMEM((NSEG, DIM), jnp.float32),
        pltpu.VMEM((NSEG, DIM), jnp.float32),
    ],
)


def kernel(x, cu_seqlens):
    cu = cu_seqlens.astype(jnp.int32)
    cu32 = jnp.concatenate(
        [cu, jnp.full((CU_PAD - NSEG - 1,), TOK, jnp.int32)])
    psum, psq = _sc_partials(x, cu32)
    tps, tpq = _tc_partials(cu32, x)
    counts = jnp.maximum(
        (cu[1:] - cu[:-1]).astype(jnp.float32), 1.0).reshape(NSEG, 1)
    return _tc_norm(cu32, counts, psum, psq, tps, tpq, x)


# speedup vs baseline: 1.2628x; 1.0099x over previous
import functools

import jax
import jax.numpy as jnp
from jax import lax
from jax.experimental import pallas as pl
from jax.experimental.pallas import tpu as pltpu
from jax.experimental.pallas import tpu_sc as plsc

TOK = 32768
DIM = 256
NSEG = 16
EPS = 1e-5

NW = 32
SC_TOK = 8192
CHUNK = SC_TOK // NW
BLK = 64
BLKP = BLK + 8
LANES = 16
NSTRIP = DIM // LANES
CU_PAD = 32


def _sc_body(x_hbm, cu_hbm, psum_hbm, psq_hbm, xbuf0, xbuf1, asum, asq,
             cu_v, sem0, sem1):
    wid = lax.axis_index("s") * 2 + lax.axis_index("c")
    base = wid * CHUNK
    pltpu.sync_copy(cu_hbm, cu_v)
    zeros = jnp.zeros((LANES,), jnp.float32)

    def cu_at(i):
        return cu_v[pl.ds(i, LANES)][0]

    def zero_body(s, carry):
        for k in range(NSTRIP):
            sl = pl.ds(k * LANES, LANES)
            asum[s, sl] = zeros
            asq[s, sl] = zeros
        return carry

    lax.fori_loop(0, NSEG, zero_body, jnp.int32(0))

    limit = base + CHUNK

    def seg_body(s, carry0):
        lo = jnp.maximum(cu_at(s), base)
        hi = jnp.minimum(cu_at(s + 1), limit)
        seg_len = jnp.maximum(hi - lo, 0)
        nblk = (seg_len + BLK - 1) >> 6

        def window(j):
            blk_lo = lo + j * BLK
            start = jnp.minimum(jnp.bitwise_and(blk_lo, -8), TOK - BLKP)
            return pl.multiple_of(start, 8)

        def accumulate(buf, j):
            blk_lo = lo + j * BLK
            start = window(j)
            loc_lo = blk_lo - start
            loc_hi = jnp.minimum(hi, blk_lo + BLK) - start

            def tok_body(t, c):
                out = []
                for k in range(NSTRIP):
                    row = buf[t, pl.ds(k * LANES, LANES)]
                    out.append(c[2 * k] + row)
                    out.append(c[2 * k + 1] + row * row)
                return tuple(out)

            acc = lax.fori_loop(loc_lo, jnp.maximum(loc_hi, loc_lo),
                                tok_body, (zeros,) * (2 * NSTRIP))
            for k in range(NSTRIP):
                sl = pl.ds(k * LANES, LANES)
                asum[s, sl] = asum[s, sl] + acc[2 * k]
                asq[s, sl] = asq[s, sl] + acc[2 * k + 1]

        def fetch(j, buf, sem):
            pltpu.async_copy(x_hbm.at[pl.ds(window(j), BLKP)], buf, sem)

        def wait(buf, sem):
            pltpu.make_async_copy(
                x_hbm.at[pl.ds(window(0), BLKP)], buf, sem).wait()

        @pl.when(nblk > 0)
        def _():
            fetch(0, xbuf0, sem0)
            npair = (nblk + 1) >> 1

            def pair_body(m, carry):
                j0 = 2 * m
                wait(xbuf0, sem0)
                fetch(jnp.minimum(j0 + 1, nblk - 1), xbuf1, sem1)
                accumulate(xbuf0, j0)
                wait(xbuf1, sem1)
                fetch(jnp.minimum(j0 + 2, nblk - 1), xbuf0, sem0)
                accumulate(xbuf1, j0 + 1)
                return carry

            lax.fori_loop(0, npair, pair_body, jnp.int32(0))
            wait(xbuf0, sem0)

        return carry0

    lax.fori_loop(0, NSEG, seg_body, jnp.int32(0))
    pltpu.sync_copy(asum, psum_hbm.at[wid])
    pltpu.sync_copy(asq, psq_hbm.at[wid])


_sc_partials = functools.partial(
    pl.kernel,
    out_type=(
        jax.ShapeDtypeStruct((NW, NSEG, DIM), jnp.float32),
        jax.ShapeDtypeStruct((NW, NSEG, DIM), jnp.float32),
    ),
    mesh=plsc.VectorSubcoreMesh(
        core_axis_name="c", subcore_axis_name="s",
        num_cores=2, num_subcores=16),
    scratch_types=[
        pltpu.VMEM((BLKP, DIM), jnp.float32),
        pltpu.VMEM((BLKP, DIM), jnp.float32),
        pltpu.VMEM((NSEG, DIM), jnp.float32),
        pltpu.VMEM((NSEG, DIM), jnp.float32),
        pltpu.VMEM((CU_PAD,), jnp.int32),
        pltpu.SemaphoreType.DMA,
        pltpu.SemaphoreType.DMA,
    ],
)(_sc_body)


PBLK = 4096
PGRID = (TOK - SC_TOK) // PBLK


def _seg_onehot(cu_smem, tok0, n):
    tok = tok0 + lax.broadcasted_iota(jnp.int32, (n, 1), 0)
    bid = jnp.zeros((n, 1), jnp.int32)
    for j in range(1, NSEG):
        bid += (tok >= cu_smem[j]).astype(jnp.int32)
    seg = lax.broadcasted_iota(jnp.int32, (1, NSEG), 1)
    return (bid == seg).astype(jnp.float32)


def _tc_part_body(cu_smem, x_ref, ps_ref, pq_ref, acc_s, acc_q):
    pid = pl.program_id(0)
    oh = _seg_onehot(cu_smem, SC_TOK + pid * PBLK, PBLK)
    xs = x_ref[...]
    dn = (((0,), (0,)), ((), ()))
    ps = lax.dot_general(oh, xs, dn, preferred_element_type=jnp.float32)
    pq = lax.dot_general(oh, xs * xs, dn,
                         preferred_element_type=jnp.float32)

    @pl.when(pid == 0)
    def _():
        acc_s[...] = ps
        acc_q[...] = pq

    @pl.when(pid > 0)
    def _():
        acc_s[...] = acc_s[...] + ps
        acc_q[...] = acc_q[...] + pq

    @pl.when(pid == PGRID - 1)
    def _():
        ps_ref[...] = acc_s[...]
        pq_ref[...] = acc_q[...]


_tc_partials = pl.pallas_call(
    _tc_part_body,
    grid=(PGRID,),
    in_specs=[
        pl.BlockSpec(memory_space=pltpu.SMEM),
        pl.BlockSpec((PBLK, DIM), lambda i: (i + SC_TOK // PBLK, 0)),
    ],
    out_specs=(
        pl.BlockSpec((NSEG, DIM), lambda i: (0, 0)),
        pl.BlockSpec((NSEG, DIM), lambda i: (0, 0)),
    ),
    out_shape=(
        jax.ShapeDtypeStruct((NSEG, DIM), jnp.float32),
        jax.ShapeDtypeStruct((NSEG, DIM), jnp.float32),
    ),
    scratch_shapes=[
        pltpu.VMEM((NSEG, DIM), jnp.float32),
        pltpu.VMEM((NSEG, DIM), jnp.float32),
    ],
)


TBLK = 4096
GRID = TOK // TBLK


def _tc_body(cu_smem, psum_ref, psq_ref, tps_ref, tpq_ref,
             x_ref, o_ref, mean_sc, rstd_sc):
    pid = pl.program_id(0)

    @pl.when(pid == 0)
    def _():
        s = jnp.sum(psum_ref[...], axis=0) + tps_ref[...]
        q = jnp.sum(psq_ref[...], axis=0) + tpq_ref[...]
        cnt = jnp.stack(
            [jnp.maximum(cu_smem[j + 1] - cu_smem[j], 1)
             for j in range(NSEG)]).astype(jnp.float32).reshape(NSEG, 1)
        mean = s / cnt
        var = jnp.maximum(q / cnt - mean * mean, 0.0)
        mean_sc[...] = mean
        rstd_sc[...] = lax.rsqrt(var + EPS)

    oh = _seg_onehot(cu_smem, pid * TBLK, TBLK)
    mu = jnp.dot(oh, mean_sc[...], preferred_element_type=jnp.float32)
    rs = jnp.dot(oh, rstd_sc[...], preferred_element_type=jnp.float32)
    o_ref[...] = (x_ref[...] - mu) * rs


_tc_norm = pl.pallas_call(
    _tc_body,
    grid=(GRID,),
    in_specs=[
        pl.BlockSpec(memory_space=pltpu.SMEM),
        pl.BlockSpec((NW, NSEG, DIM), lambda i: (0, 0, 0)),
        pl.BlockSpec((NW, NSEG, DIM), lambda i: (0, 0, 0)),
        pl.BlockSpec((NSEG, DIM), lambda i: (0, 0)),
        pl.BlockSpec((NSEG, DIM), lambda i: (0, 0)),
        pl.BlockSpec((TBLK, DIM), lambda i: (i, 0)),
    ],
    out_specs=pl.BlockSpec((TBLK, DIM), lambda i: (i, 0)),
    out_shape=jax.ShapeDtypeStruct((TOK, DIM), jnp.float32),
    scratch_shapes=[
        pltpu.VMEM((NSEG, DIM), jnp.float32),
        pltpu.VMEM((NSEG, DIM), jnp.float32),
    ],
)


def kernel(x, cu_seqlens):
    cu = cu_seqlens.astype(jnp.int32)
    cu32 = jnp.concatenate(
        [cu, jnp.full((CU_PAD - NSEG - 1,), TOK, jnp.int32)])
    psum, psq = _sc_partials(x, cu32)
    tps, tpq = _tc_partials(cu32, x)
    return _tc_norm(cu32, psum, psq, tps, tpq, x)


# speedup vs baseline: 1.2641x; 1.0010x over previous
import functools

import jax
import jax.numpy as jnp
from jax import lax
from jax.experimental import pallas as pl
from jax.experimental.pallas import tpu as pltpu
from jax.experimental.pallas import tpu_sc as plsc

TOK = 32768
DIM = 256
NSEG = 16
EPS = 1e-5

NW = 32
SC_TOK = 8192
CHUNK = SC_TOK // NW
BLK = 64
BLKP = BLK + 8
LANES = 16
NSTRIP = DIM // LANES
CU_PAD = 32


def _sc_body(x_hbm, cu_hbm, psum_hbm, psq_hbm, xbuf0, xbuf1, asum, asq,
             cu_v, sem0, sem1):
    wid = lax.axis_index("s") * 2 + lax.axis_index("c")
    base = wid * CHUNK
    pltpu.async_copy(cu_hbm, cu_v, sem0)
    zeros = jnp.zeros((LANES,), jnp.float32)

    def cu_at(i):
        return cu_v[pl.ds(i, LANES)][0]

    def zero_body(s, carry):
        for k in range(NSTRIP):
            sl = pl.ds(k * LANES, LANES)
            asum[s, sl] = zeros
            asq[s, sl] = zeros
        return carry

    lax.fori_loop(0, NSEG, zero_body, jnp.int32(0))
    pltpu.make_async_copy(cu_hbm, cu_v, sem0).wait()

    limit = base + CHUNK

    def seg_body(s, carry0):
        lo = jnp.maximum(cu_at(s), base)
        hi = jnp.minimum(cu_at(s + 1), limit)
        seg_len = jnp.maximum(hi - lo, 0)
        nblk = (seg_len + BLK - 1) >> 6

        def window(j):
            blk_lo = lo + j * BLK
            start = jnp.minimum(jnp.bitwise_and(blk_lo, -8), TOK - BLKP)
            return pl.multiple_of(start, 8)

        def accumulate(buf, j):
            blk_lo = lo + j * BLK
            start = window(j)
            loc_lo = blk_lo - start
            loc_hi = jnp.minimum(hi, blk_lo + BLK) - start

            def tok_body(t, c):
                out = []
                for k in range(NSTRIP):
                    row = buf[t, pl.ds(k * LANES, LANES)]
                    out.append(c[2 * k] + row)
                    out.append(c[2 * k + 1] + row * row)
                return tuple(out)

            acc = lax.fori_loop(loc_lo, jnp.maximum(loc_hi, loc_lo),
                                tok_body, (zeros,) * (2 * NSTRIP))
            for k in range(NSTRIP):
                sl = pl.ds(k * LANES, LANES)
                asum[s, sl] = asum[s, sl] + acc[2 * k]
                asq[s, sl] = asq[s, sl] + acc[2 * k + 1]

        def fetch(j, buf, sem):
            pltpu.async_copy(x_hbm.at[pl.ds(window(j), BLKP)], buf, sem)

        def wait(buf, sem):
            pltpu.make_async_copy(
                x_hbm.at[pl.ds(window(0), BLKP)], buf, sem).wait()

        @pl.when(nblk > 0)
        def _():
            fetch(0, xbuf0, sem0)
            npair = (nblk + 1) >> 1

            def pair_body(m, carry):
                j0 = 2 * m
                wait(xbuf0, sem0)
                fetch(jnp.minimum(j0 + 1, nblk - 1), xbuf1, sem1)
                accumulate(xbuf0, j0)
                wait(xbuf1, sem1)
                fetch(jnp.minimum(j0 + 2, nblk - 1), xbuf0, sem0)
                accumulate(xbuf1, j0 + 1)
                return carry

            lax.fori_loop(0, npair, pair_body, jnp.int32(0))
            wait(xbuf0, sem0)

        return carry0

    lax.fori_loop(0, NSEG, seg_body, jnp.int32(0))
    pltpu.async_copy(asum, psum_hbm.at[wid], sem0)
    pltpu.async_copy(asq, psq_hbm.at[wid], sem1)
    pltpu.make_async_copy(asum, psum_hbm.at[wid], sem0).wait()
    pltpu.make_async_copy(asq, psq_hbm.at[wid], sem1).wait()


_sc_partials = functools.partial(
    pl.kernel,
    out_type=(
        jax.ShapeDtypeStruct((NW, NSEG, DIM), jnp.float32),
        jax.ShapeDtypeStruct((NW, NSEG, DIM), jnp.float32),
    ),
    mesh=plsc.VectorSubcoreMesh(
        core_axis_name="c", subcore_axis_name="s",
        num_cores=2, num_subcores=16),
    scratch_types=[
        pltpu.VMEM((BLKP, DIM), jnp.float32),
        pltpu.VMEM((BLKP, DIM), jnp.float32),
        pltpu.VMEM((NSEG, DIM), jnp.float32),
        pltpu.VMEM((NSEG, DIM), jnp.float32),
        pltpu.VMEM((CU_PAD,), jnp.int32),
        pltpu.SemaphoreType.DMA,
        pltpu.SemaphoreType.DMA,
    ],
)(_sc_body)


PBLK = 4096
PGRID = (TOK - SC_TOK) // PBLK


def _seg_onehot(cu_smem, tok0, n):
    tok = tok0 + lax.broadcasted_iota(jnp.int32, (n, 1), 0)
    bid = jnp.zeros((n, 1), jnp.int32)
    for j in range(1, NSEG):
        bid += (tok >= cu_smem[j]).astype(jnp.int32)
    seg = lax.broadcasted_iota(jnp.int32, (1, NSEG), 1)
    return (bid == seg).astype(jnp.float32)


def _tc_part_body(cu_smem, x_ref, ps_ref, pq_ref, acc_s, acc_q):
    pid = pl.program_id(0)
    oh = _seg_onehot(cu_smem, SC_TOK + pid * PBLK, PBLK)
    xs = x_ref[...]
    dn = (((0,), (0,)), ((), ()))
    ps = lax.dot_general(oh, xs, dn, preferred_element_type=jnp.float32)
    pq = lax.dot_general(oh, xs * xs, dn,
                         preferred_element_type=jnp.float32)

    @pl.when(pid == 0)
    def _():
        acc_s[...] = ps
        acc_q[...] = pq

    @pl.when(pid > 0)
    def _():
        acc_s[...] = acc_s[...] + ps
        acc_q[...] = acc_q[...] + pq

    @pl.when(pid == PGRID - 1)
    def _():
        ps_ref[...] = acc_s[...]
        pq_ref[...] = acc_q[...]


_tc_partials = pl.pallas_call(
    _tc_part_body,
    grid=(PGRID,),
    in_specs=[
        pl.BlockSpec(memory_space=pltpu.SMEM),
        pl.BlockSpec((PBLK, DIM), lambda i: (i + SC_TOK // PBLK, 0)),
    ],
    out_specs=(
        pl.BlockSpec((NSEG, DIM), lambda i: (0, 0)),
        pl.BlockSpec((NSEG, DIM), lambda i: (0, 0)),
    ),
    out_shape=(
        jax.ShapeDtypeStruct((NSEG, DIM), jnp.float32),
        jax.ShapeDtypeStruct((NSEG, DIM), jnp.float32),
    ),
    scratch_shapes=[
        pltpu.VMEM((NSEG, DIM), jnp.float32),
        pltpu.VMEM((NSEG, DIM), jnp.float32),
    ],
)


TBLK = 4096
GRID = TOK // TBLK


def _tc_body(cu_smem, psum_ref, psq_ref, tps_ref, tpq_ref,
             x_ref, o_ref, mean_sc, rstd_sc):
    pid = pl.program_id(0)

    @pl.when(pid == 0)
    def _():
        s = jnp.sum(psum_ref[...], axis=0) + tps_ref[...]
        q = jnp.sum(psq_ref[...], axis=0) + tpq_ref[...]
        cnt = jnp.stack(
            [jnp.maximum(cu_smem[j + 1] - cu_smem[j], 1)
             for j in range(NSEG)]).astype(jnp.float32).reshape(NSEG, 1)
        mean = s / cnt
        var = jnp.maximum(q / cnt - mean * mean, 0.0)
        mean_sc[...] = mean
        rstd_sc[...] = lax.rsqrt(var + EPS)

    oh = _seg_onehot(cu_smem, pid * TBLK, TBLK)
    mu = jnp.dot(oh, mean_sc[...], preferred_element_type=jnp.float32)
    rs = jnp.dot(oh, rstd_sc[...], preferred_element_type=jnp.float32)
    o_ref[...] = (x_ref[...] - mu) * rs


_tc_norm = pl.pallas_call(
    _tc_body,
    grid=(GRID,),
    in_specs=[
        pl.BlockSpec(memory_space=pltpu.SMEM),
        pl.BlockSpec((NW, NSEG, DIM), lambda i: (0, 0, 0)),
        pl.BlockSpec((NW, NSEG, DIM), lambda i: (0, 0, 0)),
        pl.BlockSpec((NSEG, DIM), lambda i: (0, 0)),
        pl.BlockSpec((NSEG, DIM), lambda i: (0, 0)),
        pl.BlockSpec((TBLK, DIM), lambda i: (i, 0)),
    ],
    out_specs=pl.BlockSpec((TBLK, DIM), lambda i: (i, 0)),
    out_shape=jax.ShapeDtypeStruct((TOK, DIM), jnp.float32),
    scratch_shapes=[
        pltpu.VMEM((NSEG, DIM), jnp.float32),
        pltpu.VMEM((NSEG, DIM), jnp.float32),
    ],
)


def kernel(x, cu_seqlens):
    cu = cu_seqlens.astype(jnp.int32)
    cu32 = jnp.concatenate(
        [cu, jnp.full((CU_PAD - NSEG - 1,), TOK, jnp.int32)])
    psum, psq = _sc_partials(x, cu32)
    tps, tpq = _tc_partials(cu32, x)
    return _tc_norm(cu32, psum, psq, tps, tpq, x)
